# baseline (speedup 1.0000x reference)
import functools

import jax
import jax.numpy as jnp
from jax import lax
from jax.experimental import pallas as pl
from jax.experimental.pallas import tpu as pltpu
from jax.experimental.pallas import tpu_sc as plsc

NN = 10000
EE = 320000
D_IN = 128
D_H = 256
D_OUT = 10
NG = 64
NSUB = 16
NPAD = 10240
NROW = NPAD // NSUB
CHUNK = 100
NBUF = 4
BLK = 1000
NBLK = NN // BLK
DEGW = 128


def _sc_mesh():
    return plsc.VectorSubcoreMesh(core_axis_name="c", subcore_axis_name="s",
                                  num_cores=2, num_subcores=NSUB)


def _stream_agg(x_hbm, acc, idx_w, ibufs, isems, rows, gsems, nch):
    pltpu.sync_copy(idx_w.at[0], ibufs[0])
    pltpu.sync_copy(idx_w.at[1], ibufs[1])
    pltpu.async_copy(idx_w.at[2], ibufs[2], isems[2])
    pltpu.async_copy(idx_w.at[3], ibufs[3], isems[3])
    pltpu.async_copy(x_hbm.at[ibufs[0].at[0]], rows[0], gsems[0])
    pltpu.async_copy(x_hbm.at[ibufs[1].at[0]], rows[1], gsems[1])

    def step(jj, carry):
        for bb in range(4):
            c = 4 * jj + bb
            b = bb % 2
            pltpu.make_async_copy(x_hbm.at[ibufs[bb].at[0]], rows[b],
                                  gsems[b]).wait()
            pltpu.sync_copy(rows[b], acc.at[ibufs[bb].at[1]], add=True)

            @pl.when(c + 4 < nch)
            def _():
                pltpu.async_copy(idx_w.at[c + 4], ibufs[bb], isems[bb])

            @pl.when(c + 2 < nch)
            def _():
                nxt = (bb + 2) % 4
                pltpu.make_async_copy(idx_w.at[0], ibufs[nxt],
                                      isems[nxt]).wait()
                pltpu.async_copy(x_hbm.at[ibufs[nxt].at[0]], rows[b],
                                gsems[b])
        return carry

    lax.fori_loop(0, nch // 4, step, None)


def _ring_scratch(H):
    return ([pltpu.VMEM((2, CHUNK), jnp.int32) for _ in range(4)]
            + [pltpu.SemaphoreType.DMA for _ in range(4)]
            + [pltpu.VMEM((CHUNK, H), jnp.float32) for _ in range(2)]
            + [pltpu.SemaphoreType.DMA for _ in range(2)])


@functools.cache
def _agg_kernel(H):
    EPT = EE // NSUB
    NCH = EPT // CHUNK

    def half(x_hbm, out_hbm, idx3, zeros_hbm,
             ibufs, isems, rows, gsems, acc):
        w = lax.axis_index("s")
        nrow0 = pl.multiple_of(w * NROW, NROW)
        pltpu.sync_copy(zeros_hbm.at[pl.ds(nrow0, NROW)],
                        acc.at[pl.ds(nrow0, NROW)])
        plsc.subcore_barrier()
        _stream_agg(x_hbm, acc, idx3.at[w], ibufs, isems, rows, gsems, NCH)
        plsc.subcore_barrier()
        pltpu.sync_copy(acc.at[pl.ds(nrow0, NROW)],
                        out_hbm.at[pl.ds(nrow0, NROW)])

    @functools.partial(
        pl.kernel,
        mesh=_sc_mesh(),
        out_type=(jax.ShapeDtypeStruct((NPAD, H), jnp.float32),
                  jax.ShapeDtypeStruct((NPAD, H), jnp.float32)),
        scratch_types=_ring_scratch(H) + [
            pltpu.VMEM_SHARED((NPAD, H), jnp.float32),
        ],
    )
    def agg(xL, xR, idx3, zeros, outL, outR, *rest):
        ibufs, isems = rest[0:4], rest[4:8]
        rows, gsems, acc = rest[8:10], rest[10:12], rest[-1]
        c = lax.axis_index("c")

        @pl.when(c == 0)
        def _():
            half(xL, outL, idx3, zeros, ibufs, isems, rows, gsems, acc)

        @pl.when(c == 1)
        def _():
            half(xR, outR, idx3, zeros, ibufs, isems, rows, gsems, acc)

    return agg


@functools.cache
def _agg0_kernel():
    H = D_IN
    EPC = EE // 2
    EPT = EPC // NSUB
    NCH = EPT // CHUNK

    @functools.partial(
        pl.kernel,
        mesh=_sc_mesh(),
        out_type=(jax.ShapeDtypeStruct((NPAD, H), jnp.float32),
                  jax.ShapeDtypeStruct((NPAD, H), jnp.float32)),
        scratch_types=_ring_scratch(H) + [
            pltpu.VMEM_SHARED((NPAD, H), jnp.float32),
        ],
    )
    def agg0(x, idx4, zeros, outA, outB, *rest):
        ibufs, isems = rest[0:4], rest[4:8]
        rows, gsems, acc = rest[8:10], rest[10:12], rest[-1]
        c = lax.axis_index("c")
        w = lax.axis_index("s")
        nrow0 = pl.multiple_of(w * NROW, NROW)
        pltpu.sync_copy(zeros.at[pl.ds(nrow0, NROW)],
                        acc.at[pl.ds(nrow0, NROW)])
        plsc.subcore_barrier()
        _stream_agg(x, acc, idx4.at[c, w], ibufs, isems, rows, gsems, NCH)
        plsc.subcore_barrier()

        @pl.when(c == 0)
        def _():
            pltpu.sync_copy(acc.at[pl.ds(nrow0, NROW)],
                            outA.at[pl.ds(nrow0, NROW)])

        @pl.when(c == 1)
        def _():
            pltpu.sync_copy(acc.at[pl.ds(nrow0, NROW)],
                            outB.at[pl.ds(nrow0, NROW)])

    return agg0


@functools.cache
def _deg_kernel():
    EPC = EE // 2
    EPT = EPC // NSUB
    NCH = EPT // CHUNK

    @functools.partial(
        pl.kernel,
        mesh=_sc_mesh(),
        out_type=(jax.ShapeDtypeStruct((NPAD, DEGW), jnp.float32),
                  jax.ShapeDtypeStruct((NPAD, DEGW), jnp.float32)),
        scratch_types=[
            pltpu.VMEM((NCH, CHUNK), jnp.int32),
            pltpu.VMEM((CHUNK, DEGW), jnp.float32),
            pltpu.VMEM_SHARED((NPAD, DEGW), jnp.float32),
        ],
    )
    def deg(dst4d, ones_hbm, zeros_hbm, outA, outB, dst2d, ones_v, acc):
        c = lax.axis_index("c")
        w = lax.axis_index("s")
        nrow0 = pl.multiple_of(w * NROW, NROW)
        pltpu.sync_copy(zeros_hbm.at[pl.ds(nrow0, NROW)],
                        acc.at[pl.ds(nrow0, NROW)])
        pltpu.sync_copy(ones_hbm, ones_v)
        pltpu.sync_copy(dst4d.at[c, w], dst2d)
        plsc.subcore_barrier()

        def step(i, carry):
            pltpu.sync_copy(ones_v, acc.at[dst2d.at[i]], add=True)
            return carry

        lax.fori_loop(0, NCH, step, None)
        plsc.subcore_barrier()

        @pl.when(c == 0)
        def _():
            pltpu.sync_copy(acc.at[pl.ds(nrow0, NROW)],
                            outA.at[pl.ds(nrow0, NROW)])

        @pl.when(c == 1)
        def _():
            pltpu.sync_copy(acc.at[pl.ds(nrow0, NROW)],
                            outB.at[pl.ds(nrow0, NROW)])

    return deg


def _k1a(xL, xR, WrL, WrR, b):
    HX = xL.shape[1]

    def kern(xL_r, xR_r, WrL_r, WrR_r, b_r, P1_r):
        P1_r[...] = (jnp.dot(xL_r[...], WrL_r[...],
                             preferred_element_type=jnp.float32)
                     + jnp.dot(xR_r[...], WrR_r[...],
                               preferred_element_type=jnp.float32)
                     + b_r[...])

    row = lambda i: (i, 0)
    fix = lambda i: (0, 0)
    return pl.pallas_call(
        kern,
        grid=(NBLK,),
        in_specs=[
            pl.BlockSpec((BLK, HX), row), pl.BlockSpec((BLK, HX), row),
            pl.BlockSpec((HX, D_H), fix), pl.BlockSpec((HX, D_H), fix),
            pl.BlockSpec((1, D_H), fix),
        ],
        out_specs=pl.BlockSpec((BLK, D_H), row),
        out_shape=jax.ShapeDtypeStruct((NN, D_H), jnp.float32),
    )(xL, xR, WrL, WrR, b.reshape(1, D_H))


def _k1b(aggL, aggR, degA, degB, P1, WlL, WlR):
    HA = WlL.shape[0]

    def kern(aggL_r, aggR_r, degA_r, degB_r, P1_r,
             WlL_r, WlR_r, P_r, s_r, ss_r):
        i = pl.program_id(0)
        deg = jnp.maximum(degA_r[:, 0:1] + degB_r[:, 0:1], 1.0)
        inv = 1.0 / deg
        h = (jnp.dot(aggL_r[...] * inv, WlL_r[...],
                     preferred_element_type=jnp.float32)
             + jnp.dot(aggR_r[...] * inv, WlR_r[...],
                       preferred_element_type=jnp.float32)
             + P1_r[...])
        h = jnp.maximum(h, 0.0)
        P_r[...] = h

        @pl.when(i == 0)
        def _():
            s_r[...] = jnp.zeros_like(s_r)
            ss_r[...] = jnp.zeros_like(ss_r)

        s_r[...] += jnp.sum(h, axis=0, keepdims=True)
        ss_r[...] += jnp.sum(h * h, axis=0, keepdims=True)

    row = lambda i: (i, 0)
    fix = lambda i: (0, 0)
    return pl.pallas_call(
        kern,
        grid=(NBLK,),
        in_specs=[
            pl.BlockSpec((BLK, HA), row), pl.BlockSpec((BLK, HA), row),
            pl.BlockSpec((BLK, DEGW), row), pl.BlockSpec((BLK, DEGW), row),
            pl.BlockSpec((BLK, D_H), row),
            pl.BlockSpec((HA, D_H), fix), pl.BlockSpec((HA, D_H), fix),
        ],
        out_specs=[
            pl.BlockSpec((BLK, D_H), row),
            pl.BlockSpec((1, D_H), fix),
            pl.BlockSpec((1, D_H), fix),
        ],
        out_shape=[
            jax.ShapeDtypeStruct((NN, D_H), jnp.float32),
            jax.ShapeDtypeStruct((1, D_H), jnp.float32),
            jax.ShapeDtypeStruct((1, D_H), jnp.float32),
        ],
    )(aggL, aggR, degA, degB, P1, WlL, WlR)


def _k2(P, s, ss, gamma, beta):
    HD2 = D_H // 2

    def kern(P_r, s_r, ss_r, g_r, be_r, hL_r, hR_r):
        mean = s_r[...] / NN
        var = ss_r[...] / NN - mean * mean
        scale = g_r[...] * lax.rsqrt(var + 1e-5)
        shift = be_r[...] - mean * scale
        h = P_r[...] * scale + shift
        hL_r[...] = h[:, :HD2]
        hR_r[...] = h[:, HD2:]

    row = lambda i: (i, 0)
    fix = lambda i: (0, 0)
    return pl.pallas_call(
        kern,
        grid=(NBLK,),
        in_specs=[
            pl.BlockSpec((BLK, D_H), row),
            pl.BlockSpec((1, D_H), fix), pl.BlockSpec((1, D_H), fix),
            pl.BlockSpec((1, D_H), fix), pl.BlockSpec((1, D_H), fix),
        ],
        out_specs=[
            pl.BlockSpec((BLK, HD2), row),
            pl.BlockSpec((BLK, HD2), row),
        ],
        out_shape=[
            jax.ShapeDtypeStruct((NN, HD2), jnp.float32),
            jax.ShapeDtypeStruct((NN, HD2), jnp.float32),
        ],
    )(P, s, ss, gamma.reshape(1, D_H), beta.reshape(1, D_H))


def _k2_final(P, s, ss, gamma, beta, batch2d, g0s, g1s, W1, b1, W2, b2):
    def kern(P_r, s_r, ss_r, g_r, be_r, bat_r, g0_r, g1_r,
             W1_r, b1_r, W2_r, b2_r, out_r, pooled):
        i = pl.program_id(0)

        @pl.when(i == 0)
        def _():
            pooled[...] = jnp.full_like(pooled, -jnp.inf)

        mean = s_r[...] / NN
        var = ss_r[...] / NN - mean * mean
        scale = g_r[...] * lax.rsqrt(var + 1e-5)
        shift = be_r[...] - mean * scale
        h = P_r[...] * scale + shift
        ids = bat_r[...]

        def upd(g, carry):
            m = jnp.where(ids == g, h, -jnp.inf)
            mx = jnp.max(m, axis=0, keepdims=True)
            pooled[pl.ds(g, 1), :] = jnp.maximum(pooled[pl.ds(g, 1), :], mx)
            return carry

        lax.fori_loop(g0_r[i, 0], g1_r[i, 0] + 1, upd, None)

        @pl.when(i == NBLK - 1)
        def _():
            z = jnp.dot(pooled[...], W1_r[...],
                        preferred_element_type=jnp.float32) + b1_r[...]
            z = jnp.maximum(z, 0.0)
            z = jnp.dot(z, W2_r[...],
                        preferred_element_type=jnp.float32) + b2_r[...]
            z = jnp.maximum(z, 0.0)
            z = z - jnp.max(z, axis=1, keepdims=True)
            ez = jnp.exp(z)
            out_r[...] = ez / jnp.sum(ez, axis=1, keepdims=True)

    row = lambda i: (i, 0)
    fix = lambda i: (0, 0)
    smem = pl.BlockSpec(memory_space=pltpu.SMEM)
    return pl.pallas_call(
        kern,
        grid=(NBLK,),
        in_specs=[
            pl.BlockSpec((BLK, D_H), row),
            pl.BlockSpec((1, D_H), fix), pl.BlockSpec((1, D_H), fix),
            pl.BlockSpec((1, D_H), fix), pl.BlockSpec((1, D_H), fix),
            pl.BlockSpec((BLK, 1), row),
            smem, smem,
            pl.BlockSpec((D_H, 8), fix), pl.BlockSpec((1, 8), fix),
            pl.BlockSpec((8, D_OUT), fix), pl.BlockSpec((1, D_OUT), fix),
        ],
        out_specs=pl.BlockSpec((NG, D_OUT), fix),
        out_shape=jax.ShapeDtypeStruct((NG, D_OUT), jnp.float32),
        scratch_shapes=[pltpu.VMEM((NG, D_H), jnp.float32)],
    )(P, s, ss, gamma.reshape(1, D_H), beta.reshape(1, D_H), batch2d,
      g0s, g1s, W1, b1.reshape(1, 8), W2, b2.reshape(1, D_OUT))


def kernel(x, edge_index, batch,
           Wl_0, Wr_0, b_0, gamma_0, beta_0,
           Wl_1, Wr_1, b_1, gamma_1, beta_1,
           Wl_2, Wr_2, b_2, gamma_2, beta_2,
           W1, b1, W2, b2):
    src = edge_index[0]
    dst = edge_index[1]
    n12 = (EE // NSUB) // CHUNK
    n0 = (EE // 2 // NSUB) // CHUNK
    idx3 = jnp.stack([src.reshape(NSUB, n12, CHUNK),
                      dst.reshape(NSUB, n12, CHUNK)], axis=2)
    idx4 = jnp.stack([src.reshape(2, NSUB, n0, CHUNK),
                      dst.reshape(2, NSUB, n0, CHUNK)], axis=3)
    dst4d = dst.reshape(2, NSUB, n0, CHUNK)

    zeros = jnp.zeros((NPAD, DEGW), jnp.float32)
    ones = jnp.ones((CHUNK, DEGW), jnp.float32)
    degA, degB = _deg_kernel()(dst4d, ones, zeros)

    blocks = [
        (Wl_0, Wr_0, b_0, gamma_0, beta_0),
        (Wl_1, Wr_1, b_1, gamma_1, beta_1),
        (Wl_2, Wr_2, b_2, gamma_2, beta_2),
    ]

    HX = D_IN // 2
    xL, xR = x[:, :HX], x[:, HX:]
    out = None
    for i, (Wl, Wr, b, g, be) in enumerate(blocks):
        P1 = _k1a(xL, xR, Wr[:HX], Wr[HX:], b)
        if i == 0:
            aggL, aggR = _agg0_kernel()(x, idx4, zeros)
            WlL, WlR = Wl, Wl
        else:
            aggL, aggR = _agg_kernel(HX)(xL, xR, idx3, zeros)
            WlL, WlR = Wl[:HX], Wl[HX:]
        P, s, ss = _k1b(aggL, aggR, degA, degB, P1, WlL, WlR)
        if i < 2:
            xL, xR = _k2(P, s, ss, g, be)
            HX = D_H // 2
        else:
            batch2d = batch.reshape(NN, 1)
            g0s = batch[0::BLK].reshape(NBLK, 1)
            g1s = batch[BLK - 1::BLK].reshape(NBLK, 1)
            out = _k2_final(P, s, ss, g, be, batch2d, g0s, g1s,
                            W1, b1, W2, b2)
    return out


# speedup vs baseline: 1.0422x; 1.0422x over previous
import functools

import jax
import jax.numpy as jnp
from jax import lax
from jax.experimental import pallas as pl
from jax.experimental.pallas import tpu as pltpu
from jax.experimental.pallas import tpu_sc as plsc

NN = 10000
EE = 320000
D_IN = 128
D_H = 256
D_OUT = 10
NG = 64
NSUB = 16
NPAD = 10240
NROW = NPAD // NSUB
CHUNK = 125
NBUF = 4
BLK = 1000
NBLK = NN // BLK
DEGW = 128


def _sc_mesh():
    return plsc.VectorSubcoreMesh(core_axis_name="c", subcore_axis_name="s",
                                  num_cores=2, num_subcores=NSUB)


def _stream_agg(x_hbm, acc, idx_w, ibufs, isems, rows, gsems, nch):
    pltpu.sync_copy(idx_w.at[0], ibufs[0])
    pltpu.sync_copy(idx_w.at[1], ibufs[1])
    pltpu.async_copy(idx_w.at[2], ibufs[2], isems[2])
    pltpu.async_copy(idx_w.at[3], ibufs[3], isems[3])
    pltpu.async_copy(x_hbm.at[ibufs[0].at[0]], rows[0], gsems[0])
    pltpu.async_copy(x_hbm.at[ibufs[1].at[0]], rows[1], gsems[1])

    def step(jj, carry):
        for bb in range(4):
            c = 4 * jj + bb
            b = bb % 2
            pltpu.make_async_copy(x_hbm.at[ibufs[bb].at[0]], rows[b],
                                  gsems[b]).wait()
            pltpu.sync_copy(rows[b], acc.at[ibufs[bb].at[1]], add=True)

            @pl.when(c + 4 < nch)
            def _():
                pltpu.async_copy(idx_w.at[c + 4], ibufs[bb], isems[bb])

            @pl.when(c + 2 < nch)
            def _():
                nxt = (bb + 2) % 4
                pltpu.make_async_copy(idx_w.at[0], ibufs[nxt],
                                      isems[nxt]).wait()
                pltpu.async_copy(x_hbm.at[ibufs[nxt].at[0]], rows[b],
                                gsems[b])
        return carry

    lax.fori_loop(0, nch // 4, step, None)


def _ring_scratch(H):
    return ([pltpu.VMEM((2, CHUNK), jnp.int32) for _ in range(4)]
            + [pltpu.SemaphoreType.DMA for _ in range(4)]
            + [pltpu.VMEM((CHUNK, H), jnp.float32) for _ in range(2)]
            + [pltpu.SemaphoreType.DMA for _ in range(2)])


@functools.cache
def _agg_kernel(H):
    EPT = EE // NSUB
    NCH = EPT // CHUNK

    def half(x_hbm, out_hbm, idx3, zeros_hbm,
             ibufs, isems, rows, gsems, acc):
        w = lax.axis_index("s")
        nrow0 = pl.multiple_of(w * NROW, NROW)
        pltpu.sync_copy(zeros_hbm.at[pl.ds(nrow0, NROW)],
                        acc.at[pl.ds(nrow0, NROW)])
        plsc.subcore_barrier()
        _stream_agg(x_hbm, acc, idx3.at[w], ibufs, isems, rows, gsems, NCH)
        plsc.subcore_barrier()
        pltpu.sync_copy(acc.at[pl.ds(nrow0, NROW)],
                        out_hbm.at[pl.ds(nrow0, NROW)])

    @functools.partial(
        pl.kernel,
        mesh=_sc_mesh(),
        out_type=(jax.ShapeDtypeStruct((NPAD, H), jnp.float32),
                  jax.ShapeDtypeStruct((NPAD, H), jnp.float32)),
        scratch_types=_ring_scratch(H) + [
            pltpu.VMEM_SHARED((NPAD, H), jnp.float32),
        ],
    )
    def agg(xL, xR, idx3, zeros, outL, outR, *rest):
        ibufs, isems = rest[0:4], rest[4:8]
        rows, gsems, acc = rest[8:10], rest[10:12], rest[-1]
        c = lax.axis_index("c")

        @pl.when(c == 0)
        def _():
            half(xL, outL, idx3, zeros, ibufs, isems, rows, gsems, acc)

        @pl.when(c == 1)
        def _():
            half(xR, outR, idx3, zeros, ibufs, isems, rows, gsems, acc)

    return agg


@functools.cache
def _agg0_kernel():
    H = D_IN
    EPC = EE // 2
    EPT = EPC // NSUB
    NCH = EPT // CHUNK

    @functools.partial(
        pl.kernel,
        mesh=_sc_mesh(),
        out_type=(jax.ShapeDtypeStruct((NPAD, H), jnp.float32),
                  jax.ShapeDtypeStruct((NPAD, H), jnp.float32)),
        scratch_types=_ring_scratch(H) + [
            pltpu.VMEM_SHARED((NPAD, H), jnp.float32),
        ],
    )
    def agg0(x, idx4, zeros, outA, outB, *rest):
        ibufs, isems = rest[0:4], rest[4:8]
        rows, gsems, acc = rest[8:10], rest[10:12], rest[-1]
        c = lax.axis_index("c")
        w = lax.axis_index("s")
        nrow0 = pl.multiple_of(w * NROW, NROW)
        pltpu.sync_copy(zeros.at[pl.ds(nrow0, NROW)],
                        acc.at[pl.ds(nrow0, NROW)])
        plsc.subcore_barrier()
        _stream_agg(x, acc, idx4.at[c, w], ibufs, isems, rows, gsems, NCH)
        plsc.subcore_barrier()

        @pl.when(c == 0)
        def _():
            pltpu.sync_copy(acc.at[pl.ds(nrow0, NROW)],
                            outA.at[pl.ds(nrow0, NROW)])

        @pl.when(c == 1)
        def _():
            pltpu.sync_copy(acc.at[pl.ds(nrow0, NROW)],
                            outB.at[pl.ds(nrow0, NROW)])

    return agg0


@functools.cache
def _deg_kernel():
    EPC = EE // 2
    EPT = EPC // NSUB
    NCH = EPT // CHUNK

    @functools.partial(
        pl.kernel,
        mesh=_sc_mesh(),
        out_type=(jax.ShapeDtypeStruct((NPAD, DEGW), jnp.float32),
                  jax.ShapeDtypeStruct((NPAD, DEGW), jnp.float32)),
        scratch_types=[
            pltpu.VMEM((NCH, CHUNK), jnp.int32),
            pltpu.VMEM((CHUNK, DEGW), jnp.float32),
            pltpu.VMEM_SHARED((NPAD, DEGW), jnp.float32),
        ],
    )
    def deg(dst4d, ones_hbm, zeros_hbm, outA, outB, dst2d, ones_v, acc):
        c = lax.axis_index("c")
        w = lax.axis_index("s")
        nrow0 = pl.multiple_of(w * NROW, NROW)
        pltpu.sync_copy(zeros_hbm.at[pl.ds(nrow0, NROW)],
                        acc.at[pl.ds(nrow0, NROW)])
        pltpu.sync_copy(ones_hbm, ones_v)
        pltpu.sync_copy(dst4d.at[c, w], dst2d)
        plsc.subcore_barrier()

        def step(i, carry):
            pltpu.sync_copy(ones_v, acc.at[dst2d.at[i]], add=True)
            return carry

        lax.fori_loop(0, NCH, step, None)
        plsc.subcore_barrier()

        @pl.when(c == 0)
        def _():
            pltpu.sync_copy(acc.at[pl.ds(nrow0, NROW)],
                            outA.at[pl.ds(nrow0, NROW)])

        @pl.when(c == 1)
        def _():
            pltpu.sync_copy(acc.at[pl.ds(nrow0, NROW)],
                            outB.at[pl.ds(nrow0, NROW)])

    return deg


def _k1(aggL, aggR, degA, degB, xL, xR, WlL, WlR, WrL, WrR, b):
    HA = WlL.shape[0]
    HX = xL.shape[1]

    def kern(aggL_r, aggR_r, degA_r, degB_r, xL_r, xR_r,
             WlL_r, WlR_r, WrL_r, WrR_r, b_r, P_r, s_r, ss_r):
        i = pl.program_id(0)
        deg = jnp.maximum(degA_r[:, 0:1] + degB_r[:, 0:1], 1.0)
        inv = 1.0 / deg
        bf = jnp.bfloat16
        h = (jnp.dot((aggL_r[...] * inv).astype(bf), WlL_r[...],
                     preferred_element_type=jnp.float32)
             + jnp.dot((aggR_r[...] * inv).astype(bf), WlR_r[...],
                       preferred_element_type=jnp.float32)
             + jnp.dot(xL_r[...].astype(bf), WrL_r[...],
                       preferred_element_type=jnp.float32)
             + jnp.dot(xR_r[...].astype(bf), WrR_r[...],
                       preferred_element_type=jnp.float32)
             + b_r[...])
        h = jnp.maximum(h, 0.0)
        P_r[...] = h

        @pl.when(i == 0)
        def _():
            s_r[...] = jnp.zeros_like(s_r)
            ss_r[...] = jnp.zeros_like(ss_r)

        s_r[...] += jnp.sum(h, axis=0, keepdims=True)
        ss_r[...] += jnp.sum(h * h, axis=0, keepdims=True)

    row = lambda i: (i, 0)
    fix = lambda i: (0, 0)
    return pl.pallas_call(
        kern,
        grid=(NBLK,),
        in_specs=[
            pl.BlockSpec((BLK, HA), row), pl.BlockSpec((BLK, HA), row),
            pl.BlockSpec((BLK, DEGW), row), pl.BlockSpec((BLK, DEGW), row),
            pl.BlockSpec((BLK, HX), row), pl.BlockSpec((BLK, HX), row),
            pl.BlockSpec((HA, D_H), fix), pl.BlockSpec((HA, D_H), fix),
            pl.BlockSpec((HX, D_H), fix), pl.BlockSpec((HX, D_H), fix),
            pl.BlockSpec((1, D_H), fix),
        ],
        out_specs=[
            pl.BlockSpec((BLK, D_H), row),
            pl.BlockSpec((1, D_H), fix),
            pl.BlockSpec((1, D_H), fix),
        ],
        out_shape=[
            jax.ShapeDtypeStruct((NN, D_H), jnp.float32),
            jax.ShapeDtypeStruct((1, D_H), jnp.float32),
            jax.ShapeDtypeStruct((1, D_H), jnp.float32),
        ],
    )(aggL, aggR, degA, degB, xL, xR, WlL, WlR, WrL, WrR,
      b.reshape(1, D_H))


def _k2(P, s, ss, gamma, beta):
    HD2 = D_H // 2

    def kern(P_r, s_r, ss_r, g_r, be_r, hL_r, hR_r):
        mean = s_r[...] / NN
        var = ss_r[...] / NN - mean * mean
        scale = g_r[...] * lax.rsqrt(var + 1e-5)
        shift = be_r[...] - mean * scale
        h = P_r[...] * scale + shift
        hL_r[...] = h[:, :HD2]
        hR_r[...] = h[:, HD2:]

    row = lambda i: (i, 0)
    fix = lambda i: (0, 0)
    return pl.pallas_call(
        kern,
        grid=(NBLK,),
        in_specs=[
            pl.BlockSpec((BLK, D_H), row),
            pl.BlockSpec((1, D_H), fix), pl.BlockSpec((1, D_H), fix),
            pl.BlockSpec((1, D_H), fix), pl.BlockSpec((1, D_H), fix),
        ],
        out_specs=[
            pl.BlockSpec((BLK, HD2), row),
            pl.BlockSpec((BLK, HD2), row),
        ],
        out_shape=[
            jax.ShapeDtypeStruct((NN, HD2), jnp.float32),
            jax.ShapeDtypeStruct((NN, HD2), jnp.float32),
        ],
    )(P, s, ss, gamma.reshape(1, D_H), beta.reshape(1, D_H))


def _k2_final(P, s, ss, gamma, beta, batch2d, g0s, g1s, W1, b1, W2, b2):
    def kern(P_r, s_r, ss_r, g_r, be_r, bat_r, g0_r, g1_r,
             W1_r, b1_r, W2_r, b2_r, out_r, pooled):
        i = pl.program_id(0)

        @pl.when(i == 0)
        def _():
            pooled[...] = jnp.full_like(pooled, -jnp.inf)

        mean = s_r[...] / NN
        var = ss_r[...] / NN - mean * mean
        scale = g_r[...] * lax.rsqrt(var + 1e-5)
        shift = be_r[...] - mean * scale
        h = P_r[...] * scale + shift
        ids = bat_r[...]

        def upd(g, carry):
            m = jnp.where(ids == g, h, -jnp.inf)
            mx = jnp.max(m, axis=0, keepdims=True)
            pooled[pl.ds(g, 1), :] = jnp.maximum(pooled[pl.ds(g, 1), :], mx)
            return carry

        lax.fori_loop(g0_r[i, 0], g1_r[i, 0] + 1, upd, None)

        @pl.when(i == NBLK - 1)
        def _():
            z = jnp.dot(pooled[...], W1_r[...],
                        preferred_element_type=jnp.float32) + b1_r[...]
            z = jnp.maximum(z, 0.0)
            z = jnp.dot(z, W2_r[...],
                        preferred_element_type=jnp.float32) + b2_r[...]
            z = jnp.maximum(z, 0.0)
            z = z - jnp.max(z, axis=1, keepdims=True)
            ez = jnp.exp(z)
            out_r[...] = ez / jnp.sum(ez, axis=1, keepdims=True)

    row = lambda i: (i, 0)
    fix = lambda i: (0, 0)
    smem = pl.BlockSpec(memory_space=pltpu.SMEM)
    return pl.pallas_call(
        kern,
        grid=(NBLK,),
        in_specs=[
            pl.BlockSpec((BLK, D_H), row),
            pl.BlockSpec((1, D_H), fix), pl.BlockSpec((1, D_H), fix),
            pl.BlockSpec((1, D_H), fix), pl.BlockSpec((1, D_H), fix),
            pl.BlockSpec((BLK, 1), row),
            smem, smem,
            pl.BlockSpec((D_H, 8), fix), pl.BlockSpec((1, 8), fix),
            pl.BlockSpec((8, D_OUT), fix), pl.BlockSpec((1, D_OUT), fix),
        ],
        out_specs=pl.BlockSpec((NG, D_OUT), fix),
        out_shape=jax.ShapeDtypeStruct((NG, D_OUT), jnp.float32),
        scratch_shapes=[pltpu.VMEM((NG, D_H), jnp.float32)],
    )(P, s, ss, gamma.reshape(1, D_H), beta.reshape(1, D_H), batch2d,
      g0s, g1s, W1, b1.reshape(1, 8), W2, b2.reshape(1, D_OUT))


def kernel(x, edge_index, batch,
           Wl_0, Wr_0, b_0, gamma_0, beta_0,
           Wl_1, Wr_1, b_1, gamma_1, beta_1,
           Wl_2, Wr_2, b_2, gamma_2, beta_2,
           W1, b1, W2, b2):
    src = edge_index[0]
    dst = edge_index[1]
    n12 = (EE // NSUB) // CHUNK
    n0 = (EE // 2 // NSUB) // CHUNK
    idx3 = jnp.stack([src.reshape(NSUB, n12, CHUNK),
                      dst.reshape(NSUB, n12, CHUNK)], axis=2)
    idx4 = jnp.stack([src.reshape(2, NSUB, n0, CHUNK),
                      dst.reshape(2, NSUB, n0, CHUNK)], axis=3)
    dst4d = dst.reshape(2, NSUB, n0, CHUNK)

    zeros = jnp.zeros((NPAD, DEGW), jnp.float32)
    ones = jnp.ones((CHUNK, DEGW), jnp.float32)
    degA, degB = _deg_kernel()(dst4d, ones, zeros)

    blocks = [
        (Wl_0, Wr_0, b_0, gamma_0, beta_0),
        (Wl_1, Wr_1, b_1, gamma_1, beta_1),
        (Wl_2, Wr_2, b_2, gamma_2, beta_2),
    ]

    HX = D_IN // 2
    xL, xR = x[:, :HX], x[:, HX:]
    out = None
    for i, (Wl, Wr, b, g, be) in enumerate(blocks):
        if i == 0:
            aggL, aggR = _agg0_kernel()(x, idx4, zeros)
            WlL, WlR = Wl, Wl
        else:
            aggL, aggR = _agg_kernel(HX)(xL, xR, idx3, zeros)
            WlL, WlR = Wl[:HX], Wl[HX:]
        bf = jnp.bfloat16
        P, s, ss = _k1(aggL, aggR, degA, degB, xL, xR,
                       WlL.astype(bf), WlR.astype(bf),
                       Wr[:HX].astype(bf), Wr[HX:].astype(bf), b)
        if i < 2:
            xL, xR = _k2(P, s, ss, g, be)
            HX = D_H // 2
        else:
            batch2d = batch.reshape(NN, 1)
            g0s = batch[0::BLK].reshape(NBLK, 1)
            g1s = batch[BLK - 1::BLK].reshape(NBLK, 1)
            out = _k2_final(P, s, ss, g, be, batch2d, g0s, g1s,
                            W1, b1, W2, b2)
    return out


# speedup vs baseline: 1.0454x; 1.0030x over previous
import functools

import jax
import jax.numpy as jnp
from jax import lax
from jax.experimental import pallas as pl
from jax.experimental.pallas import tpu as pltpu
from jax.experimental.pallas import tpu_sc as plsc

NN = 10000
EE = 320000
D_IN = 128
D_H = 256
D_OUT = 10
NG = 64
NSUB = 16
NPAD = 10240
NROW = NPAD // NSUB
CHUNK = 125
NBUF = 4
BLK = 1000
NBLK = NN // BLK
DEGW = 128


def _sc_mesh():
    return plsc.VectorSubcoreMesh(core_axis_name="c", subcore_axis_name="s",
                                  num_cores=2, num_subcores=NSUB)


def _stream_agg(x_hbm, acc, idx_w, ibufs, isems, rows, gsems, nch):
    pltpu.sync_copy(idx_w.at[0], ibufs[0])
    pltpu.sync_copy(idx_w.at[1], ibufs[1])
    pltpu.async_copy(idx_w.at[2], ibufs[2], isems[2])
    pltpu.async_copy(idx_w.at[3], ibufs[3], isems[3])
    pltpu.async_copy(x_hbm.at[ibufs[0].at[0]], rows[0], gsems[0])
    pltpu.async_copy(x_hbm.at[ibufs[1].at[0]], rows[1], gsems[1])

    def step(jj, carry):
        for bb in range(4):
            c = 4 * jj + bb
            b = bb % 2
            pltpu.make_async_copy(x_hbm.at[ibufs[bb].at[0]], rows[b],
                                  gsems[b]).wait()
            pltpu.sync_copy(rows[b], acc.at[ibufs[bb].at[1]], add=True)

            @pl.when(c + 4 < nch)
            def _():
                pltpu.async_copy(idx_w.at[c + 4], ibufs[bb], isems[bb])

            @pl.when(c + 2 < nch)
            def _():
                nxt = (bb + 2) % 4
                pltpu.make_async_copy(idx_w.at[0], ibufs[nxt],
                                      isems[nxt]).wait()
                pltpu.async_copy(x_hbm.at[ibufs[nxt].at[0]], rows[b],
                                gsems[b])
        return carry

    lax.fori_loop(0, nch // 4, step, None)


def _ring_scratch(H):
    return ([pltpu.VMEM((2, CHUNK), jnp.int32) for _ in range(4)]
            + [pltpu.SemaphoreType.DMA for _ in range(4)]
            + [pltpu.VMEM((CHUNK, H), jnp.float32) for _ in range(2)]
            + [pltpu.SemaphoreType.DMA for _ in range(2)])


@functools.cache
def _agg_kernel(H):
    EPT = EE // NSUB
    NCH = EPT // CHUNK

    def half(x_hbm, out_hbm, idx3, zeros_hbm,
             ibufs, isems, rows, gsems, acc):
        w = lax.axis_index("s")
        nrow0 = pl.multiple_of(w * NROW, NROW)
        pltpu.sync_copy(zeros_hbm.at[pl.ds(nrow0, NROW)],
                        acc.at[pl.ds(nrow0, NROW)])
        plsc.subcore_barrier()
        _stream_agg(x_hbm, acc, idx3.at[w], ibufs, isems, rows, gsems, NCH)
        plsc.subcore_barrier()
        pltpu.sync_copy(acc.at[pl.ds(nrow0, NROW)],
                        out_hbm.at[pl.ds(nrow0, NROW)])

    @functools.partial(
        pl.kernel,
        mesh=_sc_mesh(),
        out_type=(jax.ShapeDtypeStruct((NPAD, H), jnp.float32),
                  jax.ShapeDtypeStruct((NPAD, H), jnp.float32)),
        scratch_types=_ring_scratch(H) + [
            pltpu.VMEM_SHARED((NPAD, H), jnp.float32),
        ],
    )
    def agg(xL, xR, idx3, zeros, outL, outR, *rest):
        ibufs, isems = rest[0:4], rest[4:8]
        rows, gsems, acc = rest[8:10], rest[10:12], rest[-1]
        c = lax.axis_index("c")

        @pl.when(c == 0)
        def _():
            half(xL, outL, idx3, zeros, ibufs, isems, rows, gsems, acc)

        @pl.when(c == 1)
        def _():
            half(xR, outR, idx3, zeros, ibufs, isems, rows, gsems, acc)

    return agg


@functools.cache
def _agg0_kernel():
    H = D_IN
    EPC = EE // 2
    EPT = EPC // NSUB
    NCH = EPT // CHUNK

    @functools.partial(
        pl.kernel,
        mesh=_sc_mesh(),
        out_type=(jax.ShapeDtypeStruct((NPAD, H), jnp.float32),
                  jax.ShapeDtypeStruct((NPAD, H), jnp.float32)),
        scratch_types=_ring_scratch(H) + [
            pltpu.VMEM_SHARED((NPAD, H), jnp.float32),
        ],
    )
    def agg0(x, idx4, zeros, outA, outB, *rest):
        ibufs, isems = rest[0:4], rest[4:8]
        rows, gsems, acc = rest[8:10], rest[10:12], rest[-1]
        c = lax.axis_index("c")
        w = lax.axis_index("s")
        nrow0 = pl.multiple_of(w * NROW, NROW)
        pltpu.sync_copy(zeros.at[pl.ds(nrow0, NROW)],
                        acc.at[pl.ds(nrow0, NROW)])
        plsc.subcore_barrier()
        _stream_agg(x, acc, idx4.at[c, w], ibufs, isems, rows, gsems, NCH)
        plsc.subcore_barrier()

        @pl.when(c == 0)
        def _():
            pltpu.sync_copy(acc.at[pl.ds(nrow0, NROW)],
                            outA.at[pl.ds(nrow0, NROW)])

        @pl.when(c == 1)
        def _():
            pltpu.sync_copy(acc.at[pl.ds(nrow0, NROW)],
                            outB.at[pl.ds(nrow0, NROW)])

    return agg0


@functools.cache
def _deg_kernel():
    EPC = EE // 2
    EPT = EPC // NSUB
    NCH = EPT // CHUNK

    @functools.partial(
        pl.kernel,
        mesh=_sc_mesh(),
        out_type=(jax.ShapeDtypeStruct((NPAD, DEGW), jnp.float32),
                  jax.ShapeDtypeStruct((NPAD, DEGW), jnp.float32)),
        scratch_types=[
            pltpu.VMEM((NCH, CHUNK), jnp.int32),
            pltpu.VMEM((CHUNK, DEGW), jnp.float32),
            pltpu.VMEM_SHARED((NPAD, DEGW), jnp.float32),
        ],
    )
    def deg(dst4d, ones_hbm, zeros_hbm, outA, outB, dst2d, ones_v, acc):
        c = lax.axis_index("c")
        w = lax.axis_index("s")
        nrow0 = pl.multiple_of(w * NROW, NROW)
        pltpu.sync_copy(zeros_hbm.at[pl.ds(nrow0, NROW)],
                        acc.at[pl.ds(nrow0, NROW)])
        pltpu.sync_copy(ones_hbm, ones_v)
        pltpu.sync_copy(dst4d.at[c, w], dst2d)
        plsc.subcore_barrier()

        def step(i, carry):
            pltpu.sync_copy(ones_v, acc.at[dst2d.at[i]], add=True)
            return carry

        lax.fori_loop(0, NCH, step, None)
        plsc.subcore_barrier()

        @pl.when(c == 0)
        def _():
            pltpu.sync_copy(acc.at[pl.ds(nrow0, NROW)],
                            outA.at[pl.ds(nrow0, NROW)])

        @pl.when(c == 1)
        def _():
            pltpu.sync_copy(acc.at[pl.ds(nrow0, NROW)],
                            outB.at[pl.ds(nrow0, NROW)])

    return deg


def _k1(aggL, aggR, degA, degB, xL, xR, WlL, WlR, WrL, WrR, b):
    HA = WlL.shape[0]
    HX = xL.shape[1]

    def kern(aggL_r, aggR_r, degA_r, degB_r, xL_r, xR_r,
             WlL_r, WlR_r, WrL_r, WrR_r, b_r, P_r, s_r, ss_r):
        i = pl.program_id(0)
        deg = jnp.maximum(degA_r[:, 0:1] + degB_r[:, 0:1], 1.0)
        inv = 1.0 / deg
        bf = jnp.bfloat16
        h = (jnp.dot((aggL_r[...] * inv).astype(bf), WlL_r[...],
                     preferred_element_type=jnp.float32)
             + jnp.dot((aggR_r[...] * inv).astype(bf), WlR_r[...],
                       preferred_element_type=jnp.float32)
             + jnp.dot(xL_r[...].astype(bf), WrL_r[...],
                       preferred_element_type=jnp.float32)
             + jnp.dot(xR_r[...].astype(bf), WrR_r[...],
                       preferred_element_type=jnp.float32)
             + b_r[...])
        h = jnp.maximum(h, 0.0)
        P_r[...] = h

        @pl.when(i == 0)
        def _():
            s_r[...] = jnp.zeros_like(s_r)
            ss_r[...] = jnp.zeros_like(ss_r)

        s_r[...] += jnp.sum(h, axis=0, keepdims=True)
        ss_r[...] += jnp.sum(h * h, axis=0, keepdims=True)

    row = lambda i: (i, 0)
    fix = lambda i: (0, 0)
    return pl.pallas_call(
        kern,
        grid=(NBLK,),
        in_specs=[
            pl.BlockSpec((BLK, HA), row), pl.BlockSpec((BLK, HA), row),
            pl.BlockSpec((BLK, 8), row), pl.BlockSpec((BLK, 8), row),
            pl.BlockSpec((BLK, HX), row), pl.BlockSpec((BLK, HX), row),
            pl.BlockSpec((HA, D_H), fix), pl.BlockSpec((HA, D_H), fix),
            pl.BlockSpec((HX, D_H), fix), pl.BlockSpec((HX, D_H), fix),
            pl.BlockSpec((1, D_H), fix),
        ],
        out_specs=[
            pl.BlockSpec((BLK, D_H), row),
            pl.BlockSpec((1, D_H), fix),
            pl.BlockSpec((1, D_H), fix),
        ],
        out_shape=[
            jax.ShapeDtypeStruct((NN, D_H), jnp.float32),
            jax.ShapeDtypeStruct((1, D_H), jnp.float32),
            jax.ShapeDtypeStruct((1, D_H), jnp.float32),
        ],
    )(aggL, aggR, degA, degB, xL, xR, WlL, WlR, WrL, WrR,
      b.reshape(1, D_H))


def _k2(P, s, ss, gamma, beta):
    HD2 = D_H // 2

    def kern(P_r, s_r, ss_r, g_r, be_r, hL_r, hR_r):
        mean = s_r[...] / NN
        var = ss_r[...] / NN - mean * mean
        scale = g_r[...] * lax.rsqrt(var + 1e-5)
        shift = be_r[...] - mean * scale
        h = P_r[...] * scale + shift
        hL_r[...] = h[:, :HD2]
        hR_r[...] = h[:, HD2:]

    row = lambda i: (i, 0)
    fix = lambda i: (0, 0)
    return pl.pallas_call(
        kern,
        grid=(NBLK,),
        in_specs=[
            pl.BlockSpec((BLK, D_H), row),
            pl.BlockSpec((1, D_H), fix), pl.BlockSpec((1, D_H), fix),
            pl.BlockSpec((1, D_H), fix), pl.BlockSpec((1, D_H), fix),
        ],
        out_specs=[
            pl.BlockSpec((BLK, HD2), row),
            pl.BlockSpec((BLK, HD2), row),
        ],
        out_shape=[
            jax.ShapeDtypeStruct((NN, HD2), jnp.float32),
            jax.ShapeDtypeStruct((NN, HD2), jnp.float32),
        ],
    )(P, s, ss, gamma.reshape(1, D_H), beta.reshape(1, D_H))


def _k2_final(P, s, ss, gamma, beta, batch2d, g0s, g1s, W1, b1, W2, b2):
    def kern(P_r, s_r, ss_r, g_r, be_r, bat_r, g0_r, g1_r,
             W1_r, b1_r, W2_r, b2_r, out_r, pooled):
        i = pl.program_id(0)

        @pl.when(i == 0)
        def _():
            pooled[...] = jnp.full_like(pooled, -jnp.inf)

        mean = s_r[...] / NN
        var = ss_r[...] / NN - mean * mean
        scale = g_r[...] * lax.rsqrt(var + 1e-5)
        shift = be_r[...] - mean * scale
        h = P_r[...] * scale + shift
        ids = bat_r[...]

        def upd(g, carry):
            m = jnp.where(ids == g, h, -jnp.inf)
            mx = jnp.max(m, axis=0, keepdims=True)
            pooled[pl.ds(g, 1), :] = jnp.maximum(pooled[pl.ds(g, 1), :], mx)
            return carry

        lax.fori_loop(g0_r[i, 0], g1_r[i, 0] + 1, upd, None)

        @pl.when(i == NBLK - 1)
        def _():
            z = jnp.dot(pooled[...], W1_r[...],
                        preferred_element_type=jnp.float32) + b1_r[...]
            z = jnp.maximum(z, 0.0)
            z = jnp.dot(z, W2_r[...],
                        preferred_element_type=jnp.float32) + b2_r[...]
            z = jnp.maximum(z, 0.0)
            z = z - jnp.max(z, axis=1, keepdims=True)
            ez = jnp.exp(z)
            out_r[...] = ez / jnp.sum(ez, axis=1, keepdims=True)

    row = lambda i: (i, 0)
    fix = lambda i: (0, 0)
    smem = pl.BlockSpec(memory_space=pltpu.SMEM)
    return pl.pallas_call(
        kern,
        grid=(NBLK,),
        in_specs=[
            pl.BlockSpec((BLK, D_H), row),
            pl.BlockSpec((1, D_H), fix), pl.BlockSpec((1, D_H), fix),
            pl.BlockSpec((1, D_H), fix), pl.BlockSpec((1, D_H), fix),
            pl.BlockSpec((BLK, 1), row),
            smem, smem,
            pl.BlockSpec((D_H, 8), fix), pl.BlockSpec((1, 8), fix),
            pl.BlockSpec((8, D_OUT), fix), pl.BlockSpec((1, D_OUT), fix),
        ],
        out_specs=pl.BlockSpec((NG, D_OUT), fix),
        out_shape=jax.ShapeDtypeStruct((NG, D_OUT), jnp.float32),
        scratch_shapes=[pltpu.VMEM((NG, D_H), jnp.float32)],
    )(P, s, ss, gamma.reshape(1, D_H), beta.reshape(1, D_H), batch2d,
      g0s, g1s, W1, b1.reshape(1, 8), W2, b2.reshape(1, D_OUT))


def kernel(x, edge_index, batch,
           Wl_0, Wr_0, b_0, gamma_0, beta_0,
           Wl_1, Wr_1, b_1, gamma_1, beta_1,
           Wl_2, Wr_2, b_2, gamma_2, beta_2,
           W1, b1, W2, b2):
    src = edge_index[0]
    dst = edge_index[1]
    n12 = (EE // NSUB) // CHUNK
    n0 = (EE // 2 // NSUB) // CHUNK
    idx3 = jnp.stack([src.reshape(NSUB, n12, CHUNK),
                      dst.reshape(NSUB, n12, CHUNK)], axis=2)
    idx4 = jnp.stack([src.reshape(2, NSUB, n0, CHUNK),
                      dst.reshape(2, NSUB, n0, CHUNK)], axis=3)
    dst4d = dst.reshape(2, NSUB, n0, CHUNK)

    zeros = jnp.zeros((NPAD, DEGW), jnp.float32)
    ones = jnp.ones((CHUNK, DEGW), jnp.float32)
    degA, degB = _deg_kernel()(dst4d, ones, zeros)
    degA, degB = degA[:, :8], degB[:, :8]

    blocks = [
        (Wl_0, Wr_0, b_0, gamma_0, beta_0),
        (Wl_1, Wr_1, b_1, gamma_1, beta_1),
        (Wl_2, Wr_2, b_2, gamma_2, beta_2),
    ]

    HX = D_IN // 2
    xL, xR = x[:, :HX], x[:, HX:]
    out = None
    for i, (Wl, Wr, b, g, be) in enumerate(blocks):
        if i == 0:
            aggL, aggR = _agg0_kernel()(x, idx4, zeros)
            WlL, WlR = Wl, Wl
        else:
            aggL, aggR = _agg_kernel(HX)(xL, xR, idx3, zeros)
            WlL, WlR = Wl[:HX], Wl[HX:]
        bf = jnp.bfloat16
        P, s, ss = _k1(aggL, aggR, degA, degB, xL, xR,
                       WlL.astype(bf), WlR.astype(bf),
                       Wr[:HX].astype(bf), Wr[HX:].astype(bf), b)
        if i < 2:
            xL, xR = _k2(P, s, ss, g, be)
            HX = D_H // 2
        else:
            batch2d = batch.reshape(NN, 1)
            g0s = batch[0::BLK].reshape(NBLK, 1)
            g1s = batch[BLK - 1::BLK].reshape(NBLK, 1)
            out = _k2_final(P, s, ss, g, be, batch2d, g0s, g1s,
                            W1, b1, W2, b2)
    return out


# speedup vs baseline: 1.0469x; 1.0015x over previous
import functools

import jax
import jax.numpy as jnp
from jax import lax
from jax.experimental import pallas as pl
from jax.experimental.pallas import tpu as pltpu
from jax.experimental.pallas import tpu_sc as plsc

NN = 10000
EE = 320000
D_IN = 128
D_H = 256
D_OUT = 10
NG = 64
NSUB = 16
NPAD = 10240
NROW = NPAD // NSUB
CHUNK = 125
BLK = 1000
NBLK = NN // BLK
DEGW = 128


def _sc_mesh():
    return plsc.VectorSubcoreMesh(core_axis_name="c", subcore_axis_name="s",
                                  num_cores=2, num_subcores=NSUB)


def _stream_agg(x_hbm, acc, idx_w, ibufs, isems, rows, gsems, nch):
    pltpu.sync_copy(idx_w.at[0], ibufs[0])
    pltpu.sync_copy(idx_w.at[1], ibufs[1])
    pltpu.async_copy(idx_w.at[2], ibufs[2], isems[2])
    pltpu.async_copy(idx_w.at[3], ibufs[3], isems[3])
    pltpu.async_copy(x_hbm.at[ibufs[0].at[0]], rows[0], gsems[0])
    pltpu.async_copy(x_hbm.at[ibufs[1].at[0]], rows[1], gsems[1])

    def step(jj, carry):
        for bb in range(4):
            c = 4 * jj + bb
            b = bb % 2
            pltpu.make_async_copy(x_hbm.at[ibufs[bb].at[0]], rows[b],
                                  gsems[b]).wait()
            pltpu.sync_copy(rows[b], acc.at[ibufs[bb].at[1]], add=True)

            @pl.when(c + 4 < nch)
            def _():
                pltpu.async_copy(idx_w.at[c + 4], ibufs[bb], isems[bb])

            @pl.when(c + 2 < nch)
            def _():
                nxt = (bb + 2) % 4
                pltpu.make_async_copy(idx_w.at[0], ibufs[nxt],
                                      isems[nxt]).wait()
                pltpu.async_copy(x_hbm.at[ibufs[nxt].at[0]], rows[b],
                                gsems[b])
        return carry

    lax.fori_loop(0, nch // 4, step, None)


def _ring_scratch(H):
    return ([pltpu.VMEM((2, CHUNK), jnp.int32) for _ in range(4)]
            + [pltpu.SemaphoreType.DMA for _ in range(4)]
            + [pltpu.VMEM((CHUNK, H), jnp.float32) for _ in range(2)]
            + [pltpu.SemaphoreType.DMA for _ in range(2)])


@functools.cache
def _agg_kernel(H):
    EPT = EE // NSUB
    NCH = EPT // CHUNK

    def half(x_hbm, out_hbm, idx3, zeros_hbm,
             ibufs, isems, rows, gsems, acc):
        w = lax.axis_index("s")
        nrow0 = pl.multiple_of(w * NROW, NROW)
        pltpu.sync_copy(zeros_hbm.at[pl.ds(nrow0, NROW)],
                        acc.at[pl.ds(nrow0, NROW)])
        plsc.subcore_barrier()
        _stream_agg(x_hbm, acc, idx3.at[w], ibufs, isems, rows, gsems, NCH)
        plsc.subcore_barrier()
        pltpu.sync_copy(acc.at[pl.ds(nrow0, NROW)],
                        out_hbm.at[pl.ds(nrow0, NROW)])

    @functools.partial(
        pl.kernel,
        mesh=_sc_mesh(),
        out_type=(jax.ShapeDtypeStruct((NPAD, H), jnp.float32),
                  jax.ShapeDtypeStruct((NPAD, H), jnp.float32)),
        scratch_types=_ring_scratch(H) + [
            pltpu.VMEM_SHARED((NPAD, H), jnp.float32),
        ],
    )
    def agg(xL, xR, idx3, zeros, outL, outR, *rest):
        ibufs, isems = rest[0:4], rest[4:8]
        rows, gsems, acc = rest[8:10], rest[10:12], rest[-1]
        c = lax.axis_index("c")

        @pl.when(c == 0)
        def _():
            half(xL, outL, idx3, zeros, ibufs, isems, rows, gsems, acc)

        @pl.when(c == 1)
        def _():
            half(xR, outR, idx3, zeros, ibufs, isems, rows, gsems, acc)

    return agg


@functools.cache
def _agg0_kernel():
    H = D_IN
    EPC = EE // 2
    EPT = EPC // NSUB
    NCH = EPT // CHUNK

    @functools.partial(
        pl.kernel,
        mesh=_sc_mesh(),
        out_type=(jax.ShapeDtypeStruct((NPAD, H), jnp.float32),
                  jax.ShapeDtypeStruct((NPAD, H), jnp.float32)),
        scratch_types=_ring_scratch(H) + [
            pltpu.VMEM_SHARED((NPAD, H), jnp.float32),
        ],
    )
    def agg0(x, idx4, zeros, outA, outB, *rest):
        ibufs, isems = rest[0:4], rest[4:8]
        rows, gsems, acc = rest[8:10], rest[10:12], rest[-1]
        c = lax.axis_index("c")
        w = lax.axis_index("s")
        nrow0 = pl.multiple_of(w * NROW, NROW)
        pltpu.sync_copy(zeros.at[pl.ds(nrow0, NROW)],
                        acc.at[pl.ds(nrow0, NROW)])
        plsc.subcore_barrier()
        _stream_agg(x, acc, idx4.at[c, w], ibufs, isems, rows, gsems, NCH)
        plsc.subcore_barrier()

        @pl.when(c == 0)
        def _():
            pltpu.sync_copy(acc.at[pl.ds(nrow0, NROW)],
                            outA.at[pl.ds(nrow0, NROW)])

        @pl.when(c == 1)
        def _():
            pltpu.sync_copy(acc.at[pl.ds(nrow0, NROW)],
                            outB.at[pl.ds(nrow0, NROW)])

    return agg0


@functools.cache
def _deg_kernel():
    EPC = EE // 2
    EPT = EPC // NSUB
    NCH = EPT // CHUNK

    @functools.partial(
        pl.kernel,
        mesh=_sc_mesh(),
        out_type=(jax.ShapeDtypeStruct((NPAD, DEGW), jnp.float32),
                  jax.ShapeDtypeStruct((NPAD, DEGW), jnp.float32)),
        scratch_types=[
            pltpu.VMEM((NCH, CHUNK), jnp.int32),
            pltpu.VMEM((CHUNK, DEGW), jnp.float32),
            pltpu.VMEM_SHARED((NPAD, DEGW), jnp.float32),
        ],
    )
    def deg(dst4d, ones_hbm, zeros_hbm, outA, outB, dst2d, ones_v, acc):
        c = lax.axis_index("c")
        w = lax.axis_index("s")
        nrow0 = pl.multiple_of(w * NROW, NROW)
        pltpu.sync_copy(zeros_hbm.at[pl.ds(nrow0, NROW)],
                        acc.at[pl.ds(nrow0, NROW)])
        pltpu.sync_copy(ones_hbm, ones_v)
        pltpu.sync_copy(dst4d.at[c, w], dst2d)
        plsc.subcore_barrier()

        def step(i, carry):
            pltpu.sync_copy(ones_v, acc.at[dst2d.at[i]], add=True)
            return carry

        lax.fori_loop(0, NCH, step, None)
        plsc.subcore_barrier()

        @pl.when(c == 0)
        def _():
            pltpu.sync_copy(acc.at[pl.ds(nrow0, NROW)],
                            outA.at[pl.ds(nrow0, NROW)])

        @pl.when(c == 1)
        def _():
            pltpu.sync_copy(acc.at[pl.ds(nrow0, NROW)],
                            outB.at[pl.ds(nrow0, NROW)])

    return deg


def _k1(aggL, aggR, degA, degB, xL, xR, WlL, WlR, WrL, WrR, b):
    HA = WlL.shape[0]
    HX = xL.shape[1]

    def kern(aggL_r, aggR_r, degA_r, degB_r, xL_r, xR_r,
             WlL_r, WlR_r, WrL_r, WrR_r, b_r, P_r, s_r, ss_r):
        i = pl.program_id(0)
        deg = jnp.maximum(degA_r[:, 0:1] + degB_r[:, 0:1], 1.0)
        inv = 1.0 / deg
        bf = jnp.bfloat16
        h = (jnp.dot((aggL_r[...] * inv).astype(bf), WlL_r[...],
                     preferred_element_type=jnp.float32)
             + jnp.dot((aggR_r[...] * inv).astype(bf), WlR_r[...],
                       preferred_element_type=jnp.float32)
             + jnp.dot(xL_r[...].astype(bf), WrL_r[...],
                       preferred_element_type=jnp.float32)
             + jnp.dot(xR_r[...].astype(bf), WrR_r[...],
                       preferred_element_type=jnp.float32)
             + b_r[...])
        h = jnp.maximum(h, 0.0)
        P_r[...] = h

        @pl.when(i == 0)
        def _():
            s_r[...] = jnp.zeros_like(s_r)
            ss_r[...] = jnp.zeros_like(ss_r)

        s_r[...] += jnp.sum(h, axis=0, keepdims=True)
        ss_r[...] += jnp.sum(h * h, axis=0, keepdims=True)

    row = lambda i: (i, 0)
    fix = lambda i: (0, 0)
    return pl.pallas_call(
        kern,
        grid=(NBLK,),
        in_specs=[
            pl.BlockSpec((BLK, HA), row), pl.BlockSpec((BLK, HA), row),
            pl.BlockSpec((BLK, 8), row), pl.BlockSpec((BLK, 8), row),
            pl.BlockSpec((BLK, HX), row), pl.BlockSpec((BLK, HX), row),
            pl.BlockSpec((HA, D_H), fix), pl.BlockSpec((HA, D_H), fix),
            pl.BlockSpec((HX, D_H), fix), pl.BlockSpec((HX, D_H), fix),
            pl.BlockSpec((1, D_H), fix),
        ],
        out_specs=[
            pl.BlockSpec((BLK, D_H), row),
            pl.BlockSpec((1, D_H), fix),
            pl.BlockSpec((1, D_H), fix),
        ],
        out_shape=[
            jax.ShapeDtypeStruct((NN, D_H), jnp.float32),
            jax.ShapeDtypeStruct((1, D_H), jnp.float32),
            jax.ShapeDtypeStruct((1, D_H), jnp.float32),
        ],
    )(aggL, aggR, degA, degB, xL, xR, WlL, WlR, WrL, WrR,
      b.reshape(1, D_H))


def _k2(P, s, ss, gamma, beta):
    HD2 = D_H // 2

    def kern(P_r, s_r, ss_r, g_r, be_r, hL_r, hR_r):
        mean = s_r[...] / NN
        var = ss_r[...] / NN - mean * mean
        scale = g_r[...] * lax.rsqrt(var + 1e-5)
        shift = be_r[...] - mean * scale
        h = P_r[...] * scale + shift
        hL_r[...] = h[:, :HD2]
        hR_r[...] = h[:, HD2:]

    row = lambda i: (i, 0)
    fix = lambda i: (0, 0)
    return pl.pallas_call(
        kern,
        grid=(NBLK,),
        in_specs=[
            pl.BlockSpec((BLK, D_H), row),
            pl.BlockSpec((1, D_H), fix), pl.BlockSpec((1, D_H), fix),
            pl.BlockSpec((1, D_H), fix), pl.BlockSpec((1, D_H), fix),
        ],
        out_specs=[
            pl.BlockSpec((BLK, HD2), row),
            pl.BlockSpec((BLK, HD2), row),
        ],
        out_shape=[
            jax.ShapeDtypeStruct((NN, HD2), jnp.float32),
            jax.ShapeDtypeStruct((NN, HD2), jnp.float32),
        ],
    )(P, s, ss, gamma.reshape(1, D_H), beta.reshape(1, D_H))


def _k2_final(P, s, ss, gamma, beta, batch2d, g0s, g1s, W1, b1, W2, b2):
    def kern(P_r, s_r, ss_r, g_r, be_r, bat_r, g0_r, g1_r,
             W1_r, b1_r, W2_r, b2_r, out_r, pooled):
        i = pl.program_id(0)

        @pl.when(i == 0)
        def _():
            pooled[...] = jnp.full_like(pooled, -jnp.inf)

        mean = s_r[...] / NN
        var = ss_r[...] / NN - mean * mean
        scale = g_r[...] * lax.rsqrt(var + 1e-5)
        shift = be_r[...] - mean * scale
        h = P_r[...] * scale + shift
        ids = bat_r[...]

        def upd(g, carry):
            m = jnp.where(ids == g, h, -jnp.inf)
            mx = jnp.max(m, axis=0, keepdims=True)
            pooled[pl.ds(g, 1), :] = jnp.maximum(pooled[pl.ds(g, 1), :], mx)
            return carry

        lax.fori_loop(g0_r[i, 0], g1_r[i, 0] + 1, upd, None)

        @pl.when(i == NBLK - 1)
        def _():
            z = jnp.dot(pooled[...], W1_r[...],
                        preferred_element_type=jnp.float32) + b1_r[...]
            z = jnp.maximum(z, 0.0)
            z = jnp.dot(z, W2_r[...],
                        preferred_element_type=jnp.float32) + b2_r[...]
            z = jnp.maximum(z, 0.0)
            z = z - jnp.max(z, axis=1, keepdims=True)
            ez = jnp.exp(z)
            out_r[...] = ez / jnp.sum(ez, axis=1, keepdims=True)

    row = lambda i: (i, 0)
    fix = lambda i: (0, 0)
    smem = pl.BlockSpec(memory_space=pltpu.SMEM)
    return pl.pallas_call(
        kern,
        grid=(NBLK,),
        in_specs=[
            pl.BlockSpec((BLK, D_H), row),
            pl.BlockSpec((1, D_H), fix), pl.BlockSpec((1, D_H), fix),
            pl.BlockSpec((1, D_H), fix), pl.BlockSpec((1, D_H), fix),
            pl.BlockSpec((BLK, 1), row),
            smem, smem,
            pl.BlockSpec((D_H, 8), fix), pl.BlockSpec((1, 8), fix),
            pl.BlockSpec((8, D_OUT), fix), pl.BlockSpec((1, D_OUT), fix),
        ],
        out_specs=pl.BlockSpec((NG, D_OUT), fix),
        out_shape=jax.ShapeDtypeStruct((NG, D_OUT), jnp.float32),
        scratch_shapes=[pltpu.VMEM((NG, D_H), jnp.float32)],
    )(P, s, ss, gamma.reshape(1, D_H), beta.reshape(1, D_H), batch2d,
      g0s, g1s, W1, b1.reshape(1, 8), W2, b2.reshape(1, D_OUT))


def kernel(x, edge_index, batch,
           Wl_0, Wr_0, b_0, gamma_0, beta_0,
           Wl_1, Wr_1, b_1, gamma_1, beta_1,
           Wl_2, Wr_2, b_2, gamma_2, beta_2,
           W1, b1, W2, b2):
    src = edge_index[0]
    dst = edge_index[1]
    n12 = (EE // NSUB) // CHUNK
    n0 = (EE // 2 // NSUB) // CHUNK
    idx3 = jnp.stack([src.reshape(NSUB, n12, CHUNK),
                      dst.reshape(NSUB, n12, CHUNK)], axis=2)
    idx4 = jnp.stack([src.reshape(2, NSUB, n0, CHUNK),
                      dst.reshape(2, NSUB, n0, CHUNK)], axis=3)
    dst4d = dst.reshape(2, NSUB, n0, CHUNK)

    zeros = jnp.zeros((NPAD, DEGW), jnp.float32)
    ones = jnp.ones((CHUNK, DEGW), jnp.float32)
    degA, degB = _deg_kernel()(dst4d, ones, zeros)
    degA, degB = degA[:, :8], degB[:, :8]

    blocks = [
        (Wl_0, Wr_0, b_0, gamma_0, beta_0),
        (Wl_1, Wr_1, b_1, gamma_1, beta_1),
        (Wl_2, Wr_2, b_2, gamma_2, beta_2),
    ]

    HX = D_IN // 2
    xL, xR = x[:, :HX], x[:, HX:]
    out = None
    for i, (Wl, Wr, b, g, be) in enumerate(blocks):
        if i == 0:
            aggL, aggR = _agg0_kernel()(x, idx4, zeros)
            WlL, WlR = Wl, Wl
        else:
            aggL, aggR = _agg_kernel(HX)(xL, xR, idx3, zeros)
            WlL, WlR = Wl[:HX], Wl[HX:]
        bf = jnp.bfloat16
        P, s, ss = _k1(aggL, aggR, degA, degB, xL, xR,
                       WlL.astype(bf), WlR.astype(bf),
                       Wr[:HX].astype(bf), Wr[HX:].astype(bf), b)
        if i < 2:
            xL, xR = _k2(P, s, ss, g, be)
            HX = D_H // 2
        else:
            batch2d = batch.reshape(NN, 1)
            g0s = batch[0::BLK].reshape(NBLK, 1)
            g1s = batch[BLK - 1::BLK].reshape(NBLK, 1)
            out = _k2_final(P, s, ss, g, be, batch2d, g0s, g1s,
                            W1, b1, W2, b2)
    return out
